# Initial kernel scaffold; baseline (speedup 1.0000x reference)
import jax
import jax.numpy as jnp
from jax.experimental import pallas as pl


def kernel(x, edge_index, edge_weight, W1, b1, W2, b2):
    raise NotImplementedError("write your pallas kernel here")


# speedup vs baseline: 49.6336x; 49.6336x over previous
import functools

import jax
import jax.numpy as jnp
from jax import lax
from jax.experimental import pallas as pl
from jax.experimental.pallas import tpu as pltpu
from jax.experimental.pallas import tpu_sc as plsc

NN = 100000
EE = 3200000
NC = 2
NS = 16
NW = NC * NS
SUB = 128
CHUNK = 1024
NSUB = CHUNK // SUB
CPW = -(-EE // (NW * CHUNK))
EPW = CPW * CHUNK
E_PAD = EPW * NW
N_PAD = 100352
RPT = N_PAD // NS
BLK = 1024
GRID = N_PAD // BLK

_mesh = plsc.VectorSubcoreMesh(core_axis_name="c", subcore_axis_name="s")
_f32 = jnp.float32


def _zero_fill(zbuf, n):
    def z(i, _):
        zbuf[pl.ds(i * 16, 16)] = jnp.zeros((16,), _f32)
        return 0

    lax.fori_loop(0, n // 16, z, 0)


def _deg_body(dst_hbm, ew_hbm, out_hbm, acc, zbuf, dstb, ewb):
    cid = lax.axis_index("c")
    sid = lax.axis_index("s")
    w = sid * NC + cid
    _zero_fill(zbuf, RPT)
    pltpu.sync_copy(zbuf, acc.at[pl.ds(sid * RPT, RPT)])
    plsc.subcore_barrier()

    rbase = w * (EPW // SUB)

    def chunk(k, _):
        pltpu.sync_copy(dst_hbm.at[pl.ds(rbase + k * NSUB, NSUB)], dstb)
        pltpu.sync_copy(ew_hbm.at[pl.ds(rbase + k * NSUB, NSUB)], ewb)
        for j in range(NSUB):
            pltpu.sync_copy(ewb.at[j], acc.at[dstb.at[j]], add=True)
        return 0

    lax.fori_loop(0, CPW, chunk, 0)
    plsc.subcore_barrier()
    pltpu.sync_copy(acc.at[pl.ds(sid * RPT, RPT)],
                    out_hbm.at[pl.ds(cid * N_PAD + sid * RPT, RPT)])


_k_deg = functools.partial(
    pl.kernel,
    out_type=jax.ShapeDtypeStruct((NC * N_PAD,), _f32),
    mesh=_mesh,
    scratch_types=[
        pltpu.VMEM_SHARED((N_PAD,), _f32),
        pltpu.VMEM((RPT,), _f32),
        pltpu.VMEM((NSUB, SUB), jnp.int32),
        pltpu.VMEM((NSUB, SUB), _f32),
    ],
)(_deg_body)


def _agg_body(F, src_hbm, dst_hbm, ew_hbm, *rest):
    tabs_hbm = rest[0:F]
    out_hbm = rest[F]
    scr = rest[F + 1:]
    tabs = scr[0:F]
    accs = scr[F:2 * F]
    bufs = scr[2 * F:3 * F]
    zbuf, srcb, dstb, ewb, gsem, ssem = scr[3 * F:]
    cid = lax.axis_index("c")
    sid = lax.axis_index("s")
    w = sid * NC + cid

    for c in range(F):
        pltpu.sync_copy(tabs_hbm[c].at[pl.ds(sid * RPT, RPT)],
                        tabs[c].at[pl.ds(sid * RPT, RPT)])
    _zero_fill(zbuf, RPT)
    for c in range(F):
        pltpu.sync_copy(zbuf, accs[c].at[pl.ds(sid * RPT, RPT)])
    plsc.subcore_barrier()

    rbase = w * (EPW // SUB)

    def chunk(k, _):
        pltpu.sync_copy(src_hbm.at[pl.ds(rbase + k * NSUB, NSUB)], srcb)
        pltpu.sync_copy(dst_hbm.at[pl.ds(rbase + k * NSUB, NSUB)], dstb)
        pltpu.sync_copy(ew_hbm.at[pl.ds(rbase + k * NSUB, NSUB)], ewb)
        gd = []
        for j in range(NSUB):
            for c in range(F):
                gd.append(pltpu.async_copy(tabs[c].at[srcb.at[j]],
                                           bufs[c].at[j], gsem))
        for d in gd:
            d.wait()
        def m(i, _):
            jj = i // (SUB // 16)
            ii = (i % (SUB // 16)) * 16
            ev = ewb[jj, pl.ds(ii, 16)]
            for c in range(F):
                bufs[c][jj, pl.ds(ii, 16)] = bufs[c][jj, pl.ds(ii, 16)] * ev
            return 0

        lax.fori_loop(0, CHUNK // 16, m, 0)
        sd = []
        for j in range(NSUB):
            for c in range(F):
                sd.append(pltpu.async_copy(bufs[c].at[j],
                                           accs[c].at[dstb.at[j]], ssem,
                                           add=True))
        for d in sd:
            d.wait()
        return 0

    lax.fori_loop(0, CPW, chunk, 0)
    plsc.subcore_barrier()
    for c in range(F):
        pltpu.sync_copy(
            accs[c].at[pl.ds(sid * RPT, RPT)],
            out_hbm.at[pl.ds((cid * F + c) * N_PAD + sid * RPT, RPT)])


def _make_agg(F):
    return functools.partial(
        pl.kernel,
        out_type=jax.ShapeDtypeStruct((NC * F * N_PAD,), _f32),
        mesh=_mesh,
        scratch_types=(
            [pltpu.VMEM_SHARED((N_PAD,), _f32) for _ in range(2 * F)]
            + [pltpu.VMEM((NSUB, SUB), _f32) for _ in range(F)]
            + [
                pltpu.VMEM((RPT,), _f32),
                pltpu.VMEM((NSUB, SUB), jnp.int32),
                pltpu.VMEM((NSUB, SUB), jnp.int32),
                pltpu.VMEM((NSUB, SUB), _f32),
                pltpu.SemaphoreType.DMA,
                pltpu.SemaphoreType.DMA,
            ]
        ),
    )(functools.partial(_agg_body, F))


_k_agg3 = _make_agg(3)
_k_agg7 = _make_agg(7)


def _prep_body(degp_ref, xt_ref, dinv_ref, xs_ref):
    d = degp_ref[0] + degp_ref[1] + 1.0
    di = lax.rsqrt(d)
    dinv_ref[...] = di[None, :]
    xs_ref[...] = xt_ref[...] * di[None, :]


def _k_prep(degp, xt):
    return pl.pallas_call(
        _prep_body,
        grid=(GRID,),
        in_specs=[
            pl.BlockSpec((NC, BLK), lambda i: (0, i)),
            pl.BlockSpec((3, BLK), lambda i: (0, i)),
        ],
        out_specs=[
            pl.BlockSpec((1, BLK), lambda i: (0, i)),
            pl.BlockSpec((3, BLK), lambda i: (0, i)),
        ],
        out_shape=[
            jax.ShapeDtypeStruct((1, N_PAD), _f32),
            jax.ShapeDtypeStruct((3, N_PAD), _f32),
        ],
    )(degp, xt)


def _mid_body(racc_ref, xs_ref, dinv_ref, w1t_ref, b1_ref, w2t_ref, hs_ref):
    di = dinv_ref[...]
    a = (racc_ref[0] + racc_ref[1] + xs_ref[...]) * di
    h = jnp.maximum(
        jnp.dot(w1t_ref[...], a, preferred_element_type=_f32) + b1_ref[...],
        0.0)
    hs_ref[...] = jnp.dot(w2t_ref[...], h, preferred_element_type=_f32) * di


def _k_mid(racc1, xs, dinv, w1t, b1c, w2t):
    return pl.pallas_call(
        _mid_body,
        grid=(GRID,),
        in_specs=[
            pl.BlockSpec((NC, 3, BLK), lambda i: (0, 0, i)),
            pl.BlockSpec((3, BLK), lambda i: (0, i)),
            pl.BlockSpec((1, BLK), lambda i: (0, i)),
            pl.BlockSpec((16, 3), lambda i: (0, 0)),
            pl.BlockSpec((16, 1), lambda i: (0, 0)),
            pl.BlockSpec((7, 16), lambda i: (0, 0)),
        ],
        out_specs=pl.BlockSpec((7, BLK), lambda i: (0, i)),
        out_shape=jax.ShapeDtypeStruct((7, N_PAD), _f32),
    )(racc1, xs, dinv, w1t, b1c, w2t)


def _out_body(racc_ref, hs_ref, dinv_ref, b2_ref, out_ref):
    di = dinv_ref[...]
    out_ref[...] = (racc_ref[0] + racc_ref[1] + hs_ref[...]) * di + b2_ref[...]


def _k_out(racc2, hs, dinv, b2c):
    return pl.pallas_call(
        _out_body,
        grid=(GRID,),
        in_specs=[
            pl.BlockSpec((NC, 7, BLK), lambda i: (0, 0, i)),
            pl.BlockSpec((7, BLK), lambda i: (0, i)),
            pl.BlockSpec((1, BLK), lambda i: (0, i)),
            pl.BlockSpec((7, 1), lambda i: (0, 0)),
        ],
        out_specs=pl.BlockSpec((7, BLK), lambda i: (0, i)),
        out_shape=jax.ShapeDtypeStruct((7, N_PAD), _f32),
    )(racc2, hs, dinv, b2c)


def kernel(x, edge_index, edge_weight, W1, b1, W2, b2):
    src = edge_index[0]
    dst = edge_index[1]
    pad = E_PAD - EE
    src_p = jnp.concatenate([src, jnp.zeros((pad,), src.dtype)])
    dst_p = jnp.concatenate([dst, jnp.full((pad,), NN, dst.dtype)])
    ew_p = jnp.concatenate([edge_weight, jnp.zeros((pad,), _f32)])
    src_r = src_p.reshape(E_PAD // SUB, SUB)
    dst_r = dst_p.reshape(E_PAD // SUB, SUB)
    ew_r = ew_p.reshape(E_PAD // SUB, SUB)

    xt = jnp.zeros((3, N_PAD), _f32).at[:, :NN].set(x.T)
    w1t = W1.T
    w2t = W2.T
    b1c = b1.reshape(16, 1)
    b2c = b2.reshape(7, 1)

    degp = _k_deg(dst_r, ew_r).reshape(NC, N_PAD)
    dinv, xs = _k_prep(degp, xt)
    racc1 = _k_agg3(src_r, dst_r, ew_r,
                    *(xs[c] for c in range(3))).reshape(NC, 3, N_PAD)
    hs = _k_mid(racc1, xs, dinv, w1t, b1c, w2t)
    racc2 = _k_agg7(src_r, dst_r, ew_r,
                    *(hs[c] for c in range(7))).reshape(NC, 7, N_PAD)
    outt = _k_out(racc2, hs, dinv, b2c)
    return outt[:, :NN].T
